# baseline (speedup 1.0000x reference)
import jax
import jax.numpy as jnp
from jax.experimental import pallas as pl

B, H, W, C = 16, 32, 32, 384
HID = 128
HW = H * W
G = 1


def _one_image(x, w1_ref, b1_ref, w2_ref, b2_ref):
    xb = x.reshape(HW, C).astype(jnp.float8_e4m3fn)
    col = jax.lax.broadcasted_iota(jnp.int32, (HW, 1), 0) % W
    f8z = jnp.float8_e4m3fn(0)
    xl = jnp.where(col == 0, f8z, jnp.roll(xb, 1, axis=0))
    xr = jnp.where(col == W - 1, f8z, jnp.roll(xb, -1, axis=0))
    zrow = jnp.zeros((W, C), jnp.float8_e4m3fn)
    bufs = [
        jnp.concatenate([zrow, xl, zrow], axis=0),
        jnp.concatenate([zrow, xb, zrow], axis=0),
        jnp.concatenate([zrow, xr, zrow], axis=0),
    ]
    acc = jnp.zeros((HW, HID), jnp.float32)
    for ky in range(3):
        for kx in range(3):
            tap = bufs[kx][W * ky:W * ky + HW]
            acc = acc + jnp.dot(tap, w1_ref[3 * ky + kx],
                                preferred_element_type=jnp.float32)
    h = jnp.maximum(acc * (1.0 / 256.0) + b1_ref[0][None, :], 0.0
                    ).astype(jnp.bfloat16)
    w2 = w2_ref[...].astype(jnp.bfloat16)
    logits = jax.lax.dot_general(w2, h, (((1,), (1,)), ((), ())),
                                 preferred_element_type=jnp.float32)
    return jax.nn.sigmoid(logits + b2_ref[0, 0])


def _fused_kernel(x_ref, w1_ref, b1_ref, w2_ref, b2_ref, o_ref):
    for i in range(G):
        o_ref[i] = _one_image(x_ref[i], w1_ref, b1_ref, w2_ref, b2_ref)


@jax.jit
def kernel(dino_features, W1, b1, W2, b2):
    w1r = (jnp.transpose(W1, (2, 3, 1, 0)) * 256.0).reshape(
        9, C, HID).astype(jnp.float8_e4m3fn)
    out = pl.pallas_call(
        _fused_kernel,
        grid=(B // G,),
        in_specs=[
            pl.BlockSpec((G, H, W, C), lambda b: (b, 0, 0, 0)),
            pl.BlockSpec((9, C, HID), lambda b: (0, 0, 0)),
            pl.BlockSpec((1, HID), lambda b: (0, 0)),
            pl.BlockSpec((1, HID), lambda b: (0, 0)),
            pl.BlockSpec((1, 1), lambda b: (0, 0)),
        ],
        out_specs=pl.BlockSpec((G, 1, HW), lambda b: (b, 0, 0)),
        out_shape=jax.ShapeDtypeStruct((B, 1, HW), jnp.float32),
    )(dino_features, w1r, b1.reshape(1, HID), W2.reshape(1, HID),
      b2.reshape(1, 1))
    return out.reshape(B, H, W, 1)


# speedup vs baseline: 1.0549x; 1.0549x over previous
import jax
import jax.numpy as jnp
from jax.experimental import pallas as pl

B, H, W, C = 16, 32, 32, 384
HID = 128
HW = H * W
G = 2


def _one_image(x, w1_ref, b1_ref, w2_ref, b2_ref):
    xb = x.reshape(HW, C).astype(jnp.float8_e4m3fn)
    col = jax.lax.broadcasted_iota(jnp.int32, (HW, 1), 0) % W
    f8z = jnp.float8_e4m3fn(0)
    xl = jnp.where(col == 0, f8z, jnp.roll(xb, 1, axis=0))
    xr = jnp.where(col == W - 1, f8z, jnp.roll(xb, -1, axis=0))
    zrow = jnp.zeros((W, C), jnp.float8_e4m3fn)
    bufs = [
        jnp.concatenate([zrow, xl, zrow], axis=0),
        jnp.concatenate([zrow, xb, zrow], axis=0),
        jnp.concatenate([zrow, xr, zrow], axis=0),
    ]
    acc = jnp.zeros((HW, HID), jnp.float32)
    for ky in range(3):
        for kx in range(3):
            tap = bufs[kx][W * ky:W * ky + HW]
            acc = acc + jnp.dot(tap, w1_ref[3 * ky + kx],
                                preferred_element_type=jnp.float32)
    h = jnp.maximum(acc * (1.0 / 256.0) + b1_ref[0][None, :], 0.0
                    ).astype(jnp.bfloat16)
    w2 = w2_ref[...].astype(jnp.bfloat16)
    logits = jax.lax.dot_general(w2, h, (((1,), (1,)), ((), ())),
                                 preferred_element_type=jnp.float32)
    return jax.nn.sigmoid(logits + b2_ref[0, 0])


def _fused_kernel(x_ref, w1_ref, b1_ref, w2_ref, b2_ref, o_ref):
    part = (pl.program_id(0) % 4) * G
    for i in range(G):
        o_ref[part + i, :] = _one_image(x_ref[i], w1_ref, b1_ref, w2_ref,
                                        b2_ref)[0]


@jax.jit
def kernel(dino_features, W1, b1, W2, b2):
    w1r = (jnp.transpose(W1, (2, 3, 1, 0)) * 256.0).reshape(
        9, C, HID).astype(jnp.float8_e4m3fn)
    out = pl.pallas_call(
        _fused_kernel,
        grid=(B // G,),
        in_specs=[
            pl.BlockSpec((G, H, W, C), lambda b: (b, 0, 0, 0)),
            pl.BlockSpec((9, C, HID), lambda b: (0, 0, 0)),
            pl.BlockSpec((1, HID), lambda b: (0, 0)),
            pl.BlockSpec((1, HID), lambda b: (0, 0)),
            pl.BlockSpec((1, 1), lambda b: (0, 0)),
        ],
        out_specs=pl.BlockSpec((4 * G, HW), lambda b: (b // 4, 0)),
        out_shape=jax.ShapeDtypeStruct((B, HW), jnp.float32),
    )(dino_features, w1r, b1.reshape(1, HID), W2.reshape(1, HID),
      b2.reshape(1, 1))
    return out.reshape(B, H, W, 1)


# speedup vs baseline: 1.0613x; 1.0061x over previous
import jax
import jax.numpy as jnp
from jax.experimental import pallas as pl

B, H, W, C = 16, 32, 32, 384
HID = 128
HW = H * W
G = 2


def _one_image(x, w1_ref, b1_ref, w2_ref, b2_ref):
    xb = x.reshape(HW, C).astype(jnp.float8_e4m3fn)
    col = jax.lax.broadcasted_iota(jnp.int32, (HW, 1), 0) % W
    f8z = jnp.float8_e4m3fn(0)
    xl = jnp.where(col == 0, f8z, jnp.roll(xb, 1, axis=0))
    xr = jnp.where(col == W - 1, f8z, jnp.roll(xb, -1, axis=0))
    zrow = jnp.zeros((W, C), jnp.float8_e4m3fn)
    bufs = [
        jnp.concatenate([zrow, xl, zrow], axis=0),
        jnp.concatenate([zrow, xb, zrow], axis=0),
        jnp.concatenate([zrow, xr, zrow], axis=0),
    ]
    acc = jnp.zeros((HW, HID), jnp.float32)
    for ky in range(3):
        for kx in range(3):
            tap = bufs[kx][W * ky:W * ky + HW]
            acc = acc + jnp.dot(tap, w1_ref[3 * ky + kx],
                                preferred_element_type=jnp.float32)
    h = jnp.maximum(acc * (1.0 / 256.0) + b1_ref[0][None, :], 0.0
                    ).astype(jnp.bfloat16)
    w2 = w2_ref[...].astype(jnp.bfloat16)
    logits = jax.lax.dot_general(w2, h, (((1,), (1,)), ((), ())),
                                 preferred_element_type=jnp.float32)
    return jax.nn.sigmoid(logits + b2_ref[0, 0])


def _fused_kernel(x_ref, w1_ref, b1_ref, w2_ref, b2_ref, o_ref):
    for i in range(G):
        o_ref[i] = _one_image(x_ref[i], w1_ref, b1_ref, w2_ref, b2_ref)


@jax.jit
def kernel(dino_features, W1, b1, W2, b2):
    w1r = (jnp.transpose(W1, (2, 3, 1, 0)) * 256.0).reshape(
        9, C, HID).astype(jnp.float8_e4m3fn)
    out = pl.pallas_call(
        _fused_kernel,
        grid=(B // G,),
        in_specs=[
            pl.BlockSpec((G, H, W, C), lambda b: (b, 0, 0, 0)),
            pl.BlockSpec((9, C, HID), lambda b: (0, 0, 0)),
            pl.BlockSpec((1, HID), lambda b: (0, 0)),
            pl.BlockSpec((1, HID), lambda b: (0, 0)),
            pl.BlockSpec((1, 1), lambda b: (0, 0)),
        ],
        out_specs=pl.BlockSpec((G, 1, HW), lambda b: (b, 0, 0)),
        out_shape=jax.ShapeDtypeStruct((B, 1, HW), jnp.float32),
    )(dino_features, w1r, b1.reshape(1, HID), W2.reshape(1, HID),
      b2.reshape(1, 1))
    return out.reshape(B, H, W, 1)
